# Initial kernel scaffold; baseline (speedup 1.0000x reference)
import jax
import jax.numpy as jnp
from jax.experimental import pallas as pl


def kernel(peptide_feature, edge_index, edge_attr, proj_W, proj_b, edge_emb, cls_token, cls_edge, c0_W1, c0_b1, c0_W2, c0_b2, c1_W1, c1_b1, c1_W2, c1_b2):
    raise NotImplementedError("write your pallas kernel here")


# speedup vs baseline: 5.9732x; 5.9732x over previous
import functools

import jax
import jax.numpy as jnp
from jax import lax
from jax.experimental import pallas as pl
from jax.experimental.pallas import tpu as pltpu
from jax.experimental.pallas import tpu_sc as plsc

N = 10000
E = 320000
H = 128
FIN = 512
NUM_EDGE_TYPES = 100

NC = 2
NS = 16
W = 128
NWIN = 79
E_PAD = NC * NS * W * NWIN
ACC_ROWS = NS * 640

_mesh = plsc.VectorSubcoreMesh(core_axis_name="c", subcore_axis_name="s")


def _sc_aggr_body(x_hbm, src_hbm, dst_hbm, attr_hbm, emb_hbm, out_hbm,
                  sidx, aidx, didx, rows, ea, emb_sh, acc, sem, sem2):
    c = lax.axis_index("c")
    s = lax.axis_index("s")

    if True:
        @pl.when(s == 0)
        def _():
            pltpu.sync_copy(emb_hbm, rows.at[pl.ds(0, NUM_EDGE_TYPES)])
            pltpu.sync_copy(rows.at[pl.ds(0, NUM_EDGE_TYPES)], emb_sh)

        @pl.loop(0, W)
        def _(i):
            for k in range(H // 16):
                rows[i, pl.ds(k * 16, 16)] = jnp.zeros((16,), jnp.float32)

        for j in range(640 // W):
            pltpu.sync_copy(rows, acc.at[pl.ds(s * 640 + j * W, W)])

        plsc.subcore_barrier()

        chunk_base = (c * NS + s) * (NWIN * W)

        @pl.loop(0, NWIN)
        def _(w):
            base = chunk_base + w * W
            pltpu.sync_copy(src_hbm.at[pl.ds(base, W)], sidx)
            pltpu.sync_copy(attr_hbm.at[pl.ds(base, W)], aidx)
            blk = (c * NS + s) * NWIN + w
            pltpu.sync_copy(dst_hbm.at[pl.ds(blk, 1)], didx)
            cp1 = pltpu.async_copy(x_hbm.at[sidx], rows, sem)
            cp2 = pltpu.async_copy(emb_sh.at[aidx], ea, sem2)
            cp1.wait()
            cp2.wait()

            @pl.loop(0, W)
            def _(i):
                for k in range(H // 16):
                    sl = pl.ds(k * 16, 16)
                    rows[i, sl] = jnp.maximum(rows[i, sl] + ea[i, sl], 0.0)

            pltpu.sync_copy(rows, acc.at[didx.at[0]], add=True)

        plsc.subcore_barrier()

        for j in range(640 // W):
            r0 = s * 640 + j * W
            pltpu.sync_copy(acc.at[pl.ds(r0, W)], rows)
            pltpu.sync_copy(rows, out_hbm.at[c, pl.ds(r0, W)])


def _sc_aggr(x, src_pad, dst2d, attr_pad, emb):
    kern = pl.kernel(
        _sc_aggr_body,
        out_type=jax.ShapeDtypeStruct((NC, ACC_ROWS, H), jnp.float32),
        mesh=_mesh,
        scratch_types=[
            pltpu.VMEM((W,), jnp.int32),
            pltpu.VMEM((W,), jnp.int32),
            pltpu.VMEM((1, W), jnp.int32),
            pltpu.VMEM((W, H), jnp.float32),
            pltpu.VMEM((W, H), jnp.float32),
            pltpu.VMEM_SHARED((NUM_EDGE_TYPES, H), jnp.float32),
            pltpu.VMEM_SHARED((ACC_ROWS, H), jnp.float32),
            pltpu.SemaphoreType.DMA,
            pltpu.SemaphoreType.DMA,
        ],
    )
    return kern(x, src_pad, dst2d, attr_pad, emb)


def _proj_body(pf_ref, w_ref, b_ref, o_ref):
    o_ref[...] = (
        jnp.dot(pf_ref[...], w_ref[...], preferred_element_type=jnp.float32)
        + b_ref[...]
    )


def _proj(pf, w, b):
    br = 1000
    return pl.pallas_call(
        _proj_body,
        grid=(N // br,),
        in_specs=[
            pl.BlockSpec((br, FIN), lambda i: (i, 0)),
            pl.BlockSpec((FIN, H), lambda i: (0, 0)),
            pl.BlockSpec((1, H), lambda i: (0, 0)),
        ],
        out_specs=pl.BlockSpec((br, H), lambda i: (i, 0)),
        out_shape=jax.ShapeDtypeStruct((N, H), jnp.float32),
    )(pf, w, b.reshape(1, H))


def _mlp_body(x_ref, a0_ref, a1_ref, w1_ref, b1_ref, w2_ref, b2_ref, o_ref):
    h = x_ref[...] + a0_ref[0] + a1_ref[0]
    t = jnp.maximum(
        jnp.dot(h, w1_ref[...], preferred_element_type=jnp.float32)
        + b1_ref[...],
        0.0,
    )
    o_ref[...] = (
        jnp.dot(t, w2_ref[...], preferred_element_type=jnp.float32)
        + b2_ref[...]
        + x_ref[...]
    )


def _mlp(x, aggr, w1, b1, w2, b2):
    br = 1000
    return pl.pallas_call(
        _mlp_body,
        grid=(N // br,),
        in_specs=[
            pl.BlockSpec((br, H), lambda i: (i, 0)),
            pl.BlockSpec((1, br, H), lambda i: (0, i, 0)),
            pl.BlockSpec((1, br, H), lambda i: (1, i, 0)),
            pl.BlockSpec((H, H), lambda i: (0, 0)),
            pl.BlockSpec((1, H), lambda i: (0, 0)),
            pl.BlockSpec((H, H), lambda i: (0, 0)),
            pl.BlockSpec((1, H), lambda i: (0, 0)),
        ],
        out_specs=pl.BlockSpec((br, H), lambda i: (i, 0)),
        out_shape=jax.ShapeDtypeStruct((N, H), jnp.float32),
    )(x, aggr, aggr, w1, b1.reshape(1, H), w2, b2.reshape(1, H))


@functools.partial(jax.jit, static_argnums=())
def kernel(peptide_feature, edge_index, edge_attr, proj_W, proj_b, edge_emb,
           cls_token, cls_edge, c0_W1, c0_b1, c0_W2, c0_b2,
           c1_W1, c1_b1, c1_W2, c1_b2):
    del cls_token, cls_edge

    pad = E_PAD - E
    ar = jnp.arange(pad, dtype=jnp.int32)
    src_pad = jnp.concatenate([edge_index[0], ar % N])
    dst_pad = jnp.concatenate([edge_index[1], N + ar % (ACC_ROWS - N)])
    attr_pad = jnp.concatenate([edge_attr, ar % NUM_EDGE_TYPES])
    dst2d = dst_pad.reshape(E_PAD // W, W)

    x = _proj(peptide_feature, proj_W, proj_b)

    a = _sc_aggr(x, src_pad, dst2d, attr_pad, edge_emb)
    x = _mlp(x, a, c0_W1, c0_b1, c0_W2, c0_b2)

    a = _sc_aggr(x, src_pad, dst2d, attr_pad, edge_emb)
    x = _mlp(x, a, c1_W1, c1_b1, c1_W2, c1_b2)

    return x
